# Initial kernel scaffold; baseline (speedup 1.0000x reference)
import jax
import jax.numpy as jnp
from jax.experimental import pallas as pl


def kernel(node_groups, groups_size):
    raise NotImplementedError("write your pallas kernel here")


# speedup vs baseline: 1.5921x; 1.5921x over previous
import jax
import jax.numpy as jnp
from jax.experimental import pallas as pl

N = 512
G = 512


def _leroy_kernel(ng_ref, gs_ref, out_ref):
    ng = ng_ref[...]
    gs = gs_ref[...]

    len_groups = jnp.sum(ng, axis=1, keepdims=True)
    base = len_groups * len_groups.T

    coef = 1.0 / jnp.log(gs)
    A = (ng > 0.0).astype(jnp.float32)

    S = jax.lax.dot_general(
        A * coef, A,
        dimension_numbers=(((1,), (1,)), ((), ())),
        preferred_element_type=jnp.float32,
    )
    pair_scores = base * S

    node_neighbors = (pair_scores > 0.0).astype(jnp.float32)
    max_log = jnp.log(jnp.max(pair_scores) + 1.0)
    pair_scores = jnp.where(
        pair_scores > -1.0, jnp.log(pair_scores + 1.0) / max_log, 0.0
    )

    M = pair_scores * node_neighbors
    common = jax.lax.dot_general(
        M, M,
        dimension_numbers=(((1,), (1,)), ((), ())),
        preferred_element_type=jnp.float32,
    )
    out_ref[...] = jnp.nan_to_num(common)


@jax.jit
def kernel(node_groups, groups_size):
    gs2d = groups_size.reshape(1, G)
    return pl.pallas_call(
        _leroy_kernel,
        out_shape=jax.ShapeDtypeStruct((N, N), jnp.float32),
    )(node_groups, gs2d)
